# Initial kernel scaffold; baseline (speedup 1.0000x reference)
import jax
import jax.numpy as jnp
from jax.experimental import pallas as pl


def kernel(x, edge_index, W1, b1, W2, b2):
    raise NotImplementedError("write your pallas kernel here")


# speedup vs baseline: 12.2452x; 12.2452x over previous
import functools

import jax
import jax.numpy as jnp
from jax import lax
from jax.experimental import pallas as pl
from jax.experimental.pallas import tpu as pltpu
from jax.experimental.pallas import tpu_sc as plsc

N = 10000
E = 320000
D1 = 128
D2 = 16

NC = 2
NS = 16
NW = NC * NS
C = 128
K = -(-(E // NW) // C)
EPT = K * C
E_PAD = EPT * NW
NPAD = N + 8

ROWS_A = 640
ROWS_LAST = N - (NS - 1) * ROWS_A

_MESH = dict(core_axis_name="c", subcore_axis_name="s")


def _copy_slices(src_ref, dst_ref, s, src_offset=True):
    @pl.when(s < NS - 1)
    def _():
        sb = s * ROWS_A if src_offset else 0
        pltpu.sync_copy(src_ref.at[pl.ds(sb, ROWS_A)],
                        dst_ref.at[pl.ds(s * ROWS_A, ROWS_A)])

    @pl.when(s == NS - 1)
    def _():
        sb = (NS - 1) * ROWS_A if src_offset else 0
        pltpu.sync_copy(src_ref.at[pl.ds(sb, ROWS_LAST)],
                        dst_ref.at[pl.ds((NS - 1) * ROWS_A, ROWS_LAST)])


@functools.partial(
    pl.kernel,
    out_type=jax.ShapeDtypeStruct((NC, N, D1), jnp.float32),
    mesh=plsc.VectorSubcoreMesh(**_MESH),
    scratch_types=[
        pltpu.VMEM((K, C), jnp.int32),
        pltpu.VMEM((C, D1), jnp.float32),
        pltpu.VMEM_SHARED((NPAD, D1), jnp.float32),
    ],
    name="gcn_degree_sc",
)
def _sc_degree(dst_hbm, ones_hbm, zeros_hbm, out_hbm, idx_v, ones_v, cnt_sh):
    c = lax.axis_index("c")
    s = lax.axis_index("s")
    wid = c * NS + s
    pltpu.sync_copy(dst_hbm.at[wid], idx_v)
    pltpu.sync_copy(ones_hbm, ones_v)
    _copy_slices(zeros_hbm, cnt_sh, s, src_offset=False)
    plsc.subcore_barrier()

    def body(j, carry):
        pltpu.sync_copy(ones_v, cnt_sh.at[idx_v.at[j]], add=True)
        return carry

    lax.fori_loop(0, K, body, 0)
    plsc.subcore_barrier()
    _copy_slices(cnt_sh, out_hbm.at[c], s)


@functools.partial(
    pl.kernel,
    out_type=jax.ShapeDtypeStruct((NC, N, D1), jnp.float32),
    mesh=plsc.VectorSubcoreMesh(**_MESH),
    scratch_types=[
        pltpu.VMEM((K, C), jnp.int32),
        pltpu.VMEM((K, C), jnp.int32),
        pltpu.VMEM((C, D1), jnp.float32),
        pltpu.VMEM_SHARED((NPAD, D1), jnp.float32),
    ],
    name="gcn_propagate_sc",
)
def _sc_propagate(y_hbm, src_hbm, dst_hbm, out_hbm, src_v, dst_v, rows_v, z_sh):
    c = lax.axis_index("c")
    s = lax.axis_index("s")
    wid = c * NS + s
    pltpu.sync_copy(src_hbm.at[wid], src_v)
    pltpu.sync_copy(dst_hbm.at[wid], dst_v)
    _copy_slices(y_hbm, z_sh, s)
    plsc.subcore_barrier()

    def body(j, carry):
        pltpu.sync_copy(y_hbm.at[src_v.at[j]], rows_v)
        pltpu.sync_copy(rows_v, z_sh.at[dst_v.at[j]], add=True)
        return carry

    lax.fori_loop(0, K, body, 0)
    plsc.subcore_barrier()
    _copy_slices(z_sh, out_hbm.at[c], s)


BM = 1000


def _tc1_body(x_ref, w1_ref, p0_ref, p1_ref, y1_ref, d_ref):
    deg = p0_ref[:, 0:1] + p1_ref[:, 0:1] + 1.0
    d = lax.rsqrt(deg)
    xw = jnp.dot(x_ref[...], w1_ref[...], preferred_element_type=jnp.float32)
    y1_ref[...] = xw * d
    d_ref[...] = d


def _tc2_body(z0_ref, z1_ref, y1_ref, d_ref, b1_ref, g_ref):
    d = d_ref[...]
    agg = z0_ref[...] + z1_ref[...] - y1_ref[...]
    h = jnp.maximum(d * agg + b1_ref[...], 0.0)
    g_ref[...] = d * h


def _tc3_body(z0_ref, z1_ref, g_ref, d_ref, b2_ref, w2_ref, o_ref):
    s2 = d_ref[...] * (z0_ref[...] + z1_ref[...] - g_ref[...])
    o_ref[...] = jax.nn.sigmoid(
        jnp.dot(s2, w2_ref[...], preferred_element_type=jnp.float32)
        + b2_ref[...])


def kernel(x, edge_index, W1, b1, W2, b2):
    src = edge_index[0]
    dst = edge_index[1]
    pad = E_PAD - E
    src_p = jnp.concatenate([src, jnp.zeros((pad,), jnp.int32)]).reshape(NW, K, C)
    dst_p = jnp.concatenate([dst, jnp.full((pad,), N, jnp.int32)]).reshape(NW, K, C)
    ones = jnp.ones((C, D1), jnp.float32)
    zeros = jnp.zeros((ROWS_A, D1), jnp.float32)

    p = _sc_degree(dst_p, ones, zeros)

    y1, d = pl.pallas_call(
        _tc1_body,
        grid=(N // BM,),
        in_specs=[
            pl.BlockSpec((BM, D1), lambda i: (i, 0)),
            pl.BlockSpec((D1, D1), lambda i: (0, 0)),
            pl.BlockSpec((BM, D1), lambda i: (i, 0)),
            pl.BlockSpec((BM, D1), lambda i: (i, 0)),
        ],
        out_specs=[
            pl.BlockSpec((BM, D1), lambda i: (i, 0)),
            pl.BlockSpec((BM, 1), lambda i: (i, 0)),
        ],
        out_shape=[
            jax.ShapeDtypeStruct((N, D1), jnp.float32),
            jax.ShapeDtypeStruct((N, 1), jnp.float32),
        ],
    )(x, W1, p[0], p[1])

    zp1 = _sc_propagate(y1, src_p, dst_p)

    g = pl.pallas_call(
        _tc2_body,
        grid=(N // BM,),
        in_specs=[
            pl.BlockSpec((BM, D1), lambda i: (i, 0)),
            pl.BlockSpec((BM, D1), lambda i: (i, 0)),
            pl.BlockSpec((BM, D1), lambda i: (i, 0)),
            pl.BlockSpec((BM, 1), lambda i: (i, 0)),
            pl.BlockSpec((1, D1), lambda i: (0, 0)),
        ],
        out_specs=pl.BlockSpec((BM, D1), lambda i: (i, 0)),
        out_shape=jax.ShapeDtypeStruct((N, D1), jnp.float32),
    )(zp1[0], zp1[1], y1, d, b1.reshape(1, D1))

    zg = _sc_propagate(g, src_p, dst_p)

    out = pl.pallas_call(
        _tc3_body,
        grid=(N // BM,),
        in_specs=[
            pl.BlockSpec((BM, D1), lambda i: (i, 0)),
            pl.BlockSpec((BM, D1), lambda i: (i, 0)),
            pl.BlockSpec((BM, D1), lambda i: (i, 0)),
            pl.BlockSpec((BM, 1), lambda i: (i, 0)),
            pl.BlockSpec((1, D2), lambda i: (0, 0)),
            pl.BlockSpec((D1, D2), lambda i: (0, 0)),
        ],
        out_specs=pl.BlockSpec((BM, D2), lambda i: (i, 0)),
        out_shape=jax.ShapeDtypeStruct((N, D2), jnp.float32),
    )(zg[0], zg[1], g, d, b2.reshape(1, D2), W2)

    return out
